# Initial kernel scaffold; baseline (speedup 1.0000x reference)
import functools

import jax
import jax.numpy as jnp
from jax import lax
from jax.experimental import pallas as pl
from jax.experimental.pallas import tpu as pltpu
from jax.experimental.pallas import tpu_sc as plsc

N = 10000
E = 320000
D = 128

NC = 2
NS = 16
NW = NC * NS
C = 128
NCHUNK = 79
EPT = NCHUNK * C
E_PAD = EPT * NW
N_PAD = 10240
RPT = N_PAD // NS
DW = 16
BR = 500
GRID = N // BR

_MESH = plsc.VectorSubcoreMesh(
    core_axis_name="c", subcore_axis_name="s", num_cores=NC, num_subcores=NS
)


def _deg_body(src_hbm, dst_hbm, dego_hbm, degi_hbm,
              idx_vm, ones_vm, zero_vm, dego_sh, degi_sh):
    cid = lax.axis_index("c")
    sid = lax.axis_index("s")
    wid = cid * NS + sid

    one16 = jnp.full((16,), 1.0, jnp.float32)
    zero16 = jnp.zeros((16,), jnp.float32)

    def _fill(r, _):
        ones_vm[r, pl.ds(0, DW)] = one16
        zero_vm[r, pl.ds(0, DW)] = zero16
        return _

    lax.fori_loop(0, C, _fill, 0)

    for k in range(RPT // C):
        pltpu.sync_copy(zero_vm, dego_sh.at[pl.ds(sid * RPT + k * C, C)])
        pltpu.sync_copy(zero_vm, degi_sh.at[pl.ds(sid * RPT + k * C, C)])

    pltpu.sync_copy(src_hbm.at[wid], idx_vm.at[pl.ds(0, NCHUNK)])
    pltpu.sync_copy(dst_hbm.at[wid], idx_vm.at[pl.ds(NCHUNK, NCHUNK)])
    plsc.subcore_barrier()

    def _chunk(i, _):
        pltpu.sync_copy(ones_vm, dego_sh.at[idx_vm.at[i]], add=True)
        pltpu.sync_copy(ones_vm, degi_sh.at[idx_vm.at[NCHUNK + i]], add=True)
        return _

    lax.fori_loop(0, NCHUNK, _chunk, 0)
    plsc.subcore_barrier()

    sl = pl.ds(sid * RPT, RPT)
    pltpu.sync_copy(dego_sh.at[sl], dego_hbm.at[cid, sl])
    pltpu.sync_copy(degi_sh.at[sl], degi_hbm.at[cid, sl])


_deg_call = pl.kernel(
    _deg_body,
    out_type=(
        jax.ShapeDtypeStruct((NC, N_PAD, DW), jnp.float32),
        jax.ShapeDtypeStruct((NC, N_PAD, DW), jnp.float32),
    ),
    mesh=_MESH,
    scratch_types=[
        pltpu.VMEM((2 * NCHUNK, C), jnp.int32),
        pltpu.VMEM((C, DW), jnp.float32),
        pltpu.VMEM((C, DW), jnp.float32),
        pltpu.VMEM_SHARED((N_PAD, DW), jnp.float32),
        pltpu.VMEM_SHARED((N_PAD, DW), jnp.float32),
    ],
)


def _spmm_body(h_hbm, src_hbm, dst_hbm, out_hbm,
               sidx_vm, didx_vm, msg0, msg1, acc_sh, sem0, sem1):
    cid = lax.axis_index("c")
    sid = lax.axis_index("s")
    wid = cid * NS + sid

    zero16 = jnp.zeros((16,), jnp.float32)

    def _z(i, _):
        msg0[i // (D // 16), pl.ds((i % (D // 16)) * 16, 16)] = zero16
        return _

    lax.fori_loop(0, C * D // 16, _z, 0)

    for k in range(RPT // C):
        pltpu.sync_copy(msg0, acc_sh.at[pl.ds(sid * RPT + k * C, C)])

    pltpu.sync_copy(src_hbm.at[wid], sidx_vm)
    pltpu.sync_copy(dst_hbm.at[wid], didx_vm)
    plsc.subcore_barrier()

    pltpu.async_copy(h_hbm.at[sidx_vm.at[0]], msg0, sem0)
    pltpu.async_copy(h_hbm.at[sidx_vm.at[1]], msg1, sem1)

    def _step(j, _):
        i = 2 * j
        pltpu.make_async_copy(h_hbm.at[sidx_vm.at[i]], msg0, sem0).wait()
        pltpu.sync_copy(msg0, acc_sh.at[didx_vm.at[i]], add=True)

        @pl.when(i + 2 < NCHUNK)
        def _():
            pltpu.async_copy(h_hbm.at[sidx_vm.at[i + 2]], msg0, sem0)

        @pl.when(i + 1 < NCHUNK)
        def _():
            pltpu.make_async_copy(h_hbm.at[sidx_vm.at[i + 1]], msg1, sem1).wait()
            pltpu.sync_copy(msg1, acc_sh.at[didx_vm.at[i + 1]], add=True)

        @pl.when(i + 3 < NCHUNK)
        def _():
            pltpu.async_copy(h_hbm.at[sidx_vm.at[i + 3]], msg1, sem1)

        return _

    lax.fori_loop(0, (NCHUNK + 1) // 2, _step, 0)
    plsc.subcore_barrier()

    sl = pl.ds(sid * RPT, RPT)
    pltpu.sync_copy(acc_sh.at[sl], out_hbm.at[cid, sl])


_spmm_call = pl.kernel(
    _spmm_body,
    out_type=jax.ShapeDtypeStruct((NC, N_PAD, D), jnp.float32),
    mesh=_MESH,
    scratch_types=[
        pltpu.VMEM((NCHUNK, C), jnp.int32),
        pltpu.VMEM((NCHUNK, C), jnp.int32),
        pltpu.VMEM((C, D), jnp.float32),
        pltpu.VMEM((C, D), jnp.float32),
        pltpu.VMEM_SHARED((N_PAD, D), jnp.float32),
        pltpu.SemaphoreType.DMA,
        pltpu.SemaphoreType.DMA,
    ],
)


def _scale_body(feat_ref, dego_ref, o_ref):
    dg = dego_ref[0, :, 0] + dego_ref[1, :, 0]
    ns = lax.rsqrt(jnp.maximum(dg, 1.0))
    o_ref[...] = feat_ref[...] * ns[:, None]


def _mid_body(p_ref, degi_ref, dego_ref, w_ref, b_ref, o_ref):
    agg = p_ref[0] + p_ref[1]
    nd = lax.rsqrt(jnp.maximum(degi_ref[0, :, 0] + degi_ref[1, :, 0], 1.0))
    ns = lax.rsqrt(jnp.maximum(dego_ref[0, :, 0] + dego_ref[1, :, 0], 1.0))
    h = jnp.dot(agg * nd[:, None], w_ref[...],
                preferred_element_type=jnp.float32) + b_ref[...]
    o_ref[...] = h * ns[:, None]


def _fin_body(p_ref, degi_ref, w_ref, b_ref, o_ref):
    agg = p_ref[0] + p_ref[1]
    nd = lax.rsqrt(jnp.maximum(degi_ref[0, :, 0] + degi_ref[1, :, 0], 1.0))
    o_ref[...] = jnp.dot(agg * nd[:, None], w_ref[...],
                         preferred_element_type=jnp.float32) + b_ref[...]


_row_spec = pl.BlockSpec((BR, D), lambda i: (i, 0))
_deg_spec = pl.BlockSpec((NC, BR, DW), lambda i: (0, i, 0))
_part_spec = pl.BlockSpec((NC, BR, D), lambda i: (0, i, 0))
_w_spec = pl.BlockSpec((D, D), lambda i: (0, 0))
_b_spec = pl.BlockSpec((1, D), lambda i: (0, 0))

_scale_call = pl.pallas_call(
    _scale_body,
    grid=(GRID,),
    in_specs=[_row_spec, _deg_spec],
    out_specs=_row_spec,
    out_shape=jax.ShapeDtypeStruct((N_PAD, D), jnp.float32),
)

_mid_call = pl.pallas_call(
    _mid_body,
    grid=(GRID,),
    in_specs=[_part_spec, _deg_spec, _deg_spec, _w_spec, _b_spec],
    out_specs=_row_spec,
    out_shape=jax.ShapeDtypeStruct((N_PAD, D), jnp.float32),
)

_fin_call = pl.pallas_call(
    _fin_body,
    grid=(GRID,),
    in_specs=[_part_spec, _deg_spec, _w_spec, _b_spec],
    out_specs=_row_spec,
    out_shape=jax.ShapeDtypeStruct((N, D), jnp.float32),
)


def kernel(feat, edge_index, W1, b1, W2, b2):
    src = edge_index[0]
    dst = edge_index[1]
    pad = jnp.full((E_PAD - E,), N, jnp.int32)
    src3 = jnp.concatenate([src, pad]).reshape(NW, NCHUNK, C)
    dst3 = jnp.concatenate([dst, pad]).reshape(NW, NCHUNK, C)
    b1r = b1.reshape(1, D)
    b2r = b2.reshape(1, D)

    dego, degi = _deg_call(src3, dst3)
    h = _scale_call(feat, dego)
    parts = _spmm_call(h, src3, dst3)
    h1 = _mid_call(parts, degi, dego, W1, b1r)
    parts2 = _spmm_call(h1, src3, dst3)
    return _fin_call(parts2, degi, W2, b2r)


# speedup vs baseline: 3.0479x; 3.0479x over previous
import jax
import jax.numpy as jnp
from jax import lax
from jax.experimental import pallas as pl
from jax.experimental.pallas import tpu as pltpu
from jax.experimental.pallas import tpu_sc as plsc

N = 10000
E = 320000
D = 128

NC = 2
NS = 16
NW = NC * NS
C = 128
NCHUNK = 80
EPT = NCHUNK * C
E_PAD = EPT * NW
N_PAD = 10240
RPT = N_PAD // NS
BR = 512
GRID = N_PAD // BR
BRF = 400
GRIDF = N // BRF

_MESH = plsc.VectorSubcoreMesh(
    core_axis_name="c", subcore_axis_name="s", num_cores=NC, num_subcores=NS
)


def _count_body(idx_hbm, tab_hbm, idx_vm, ones_vm, zero_vm, tab_sh):
    cid = lax.axis_index("c")
    sid = lax.axis_index("s")
    wid = cid * NS + sid

    pltpu.sync_copy(idx_hbm.at[pl.ds(wid * NCHUNK, NCHUNK)], idx_vm)

    one16 = jnp.full((16,), 1.0, jnp.float32)
    zero16 = jnp.zeros((16,), jnp.float32)
    for r in range(C):
        for q in range(D // 16):
            ones_vm[r, pl.ds(q * 16, 16)] = one16
    for r in range(16):
        for q in range(D // 16):
            zero_vm[r, pl.ds(q * 16, 16)] = zero16

    for k in range(RPT // 16):
        pltpu.sync_copy(zero_vm, tab_sh.at[pl.ds(sid * RPT + k * 16, 16)])
    plsc.subcore_barrier()

    def _chunk(j, _):
        pltpu.sync_copy(ones_vm, tab_sh.at[idx_vm.at[j]], add=True)
        return _

    lax.fori_loop(0, NCHUNK, _chunk, 0)
    plsc.subcore_barrier()

    sl = pl.ds(sid * RPT, RPT)
    pltpu.sync_copy(tab_sh.at[sl], tab_hbm.at[cid, sl])


_count_call = pl.kernel(
    _count_body,
    out_type=jax.ShapeDtypeStruct((NC, N_PAD, D), jnp.float32),
    mesh=_MESH,
    scratch_types=[
        pltpu.VMEM((NCHUNK, C), jnp.int32),
        pltpu.VMEM((C, D), jnp.float32),
        pltpu.VMEM((16, D), jnp.float32),
        pltpu.VMEM_SHARED((N_PAD, D), jnp.float32),
    ],
)


def _spmm_body(h_hbm, sidx_hbm, didx_hbm, out_hbm,
               sidx_vm, didx_vm, msg_vm, zero_vm, acc_sh, sem):
    cid = lax.axis_index("c")
    sid = lax.axis_index("s")
    wid = cid * NS + sid

    pltpu.sync_copy(sidx_hbm.at[pl.ds(wid * NCHUNK, NCHUNK)], sidx_vm)
    pltpu.sync_copy(didx_hbm.at[pl.ds(wid * NCHUNK, NCHUNK)], didx_vm)

    zero16 = jnp.zeros((16,), jnp.float32)
    for r in range(16):
        for q in range(D // 16):
            zero_vm[r, pl.ds(q * 16, 16)] = zero16

    for k in range(RPT // 16):
        pltpu.sync_copy(zero_vm, acc_sh.at[pl.ds(sid * RPT + k * 16, 16)])
    plsc.subcore_barrier()

    def _chunk(j, _):
        pltpu.async_copy(h_hbm.at[sidx_vm.at[j]], msg_vm, sem).wait()
        pltpu.sync_copy(msg_vm, acc_sh.at[didx_vm.at[j]], add=True)
        return _

    lax.fori_loop(0, NCHUNK, _chunk, 0)
    plsc.subcore_barrier()

    sl = pl.ds(sid * RPT, RPT)
    pltpu.sync_copy(acc_sh.at[sl], out_hbm.at[cid, sl])


_spmm_call = pl.kernel(
    _spmm_body,
    out_type=jax.ShapeDtypeStruct((NC, N_PAD, D), jnp.float32),
    mesh=_MESH,
    scratch_types=[
        pltpu.VMEM((NCHUNK, C), jnp.int32),
        pltpu.VMEM((NCHUNK, C), jnp.int32),
        pltpu.VMEM((C, D), jnp.float32),
        pltpu.VMEM((16, D), jnp.float32),
        pltpu.VMEM_SHARED((N_PAD, D), jnp.float32),
        pltpu.SemaphoreType.DMA,
    ],
)


def _scale_body(feat_ref, dego_ref, o_ref):
    dg = dego_ref[0, :, 0] + dego_ref[1, :, 0]
    ns = lax.rsqrt(jnp.maximum(dg, 1.0))
    o_ref[...] = feat_ref[...] * ns[:, None]


def _mid_body(p_ref, degi_ref, dego_ref, w_ref, b_ref, o_ref):
    agg = p_ref[0] + p_ref[1]
    nd = lax.rsqrt(jnp.maximum(degi_ref[0, :, 0] + degi_ref[1, :, 0], 1.0))
    ns = lax.rsqrt(jnp.maximum(dego_ref[0, :, 0] + dego_ref[1, :, 0], 1.0))
    h = jnp.dot(agg * nd[:, None], w_ref[...],
                preferred_element_type=jnp.float32) + b_ref[...]
    o_ref[...] = h * ns[:, None]


def _fin_body(p_ref, degi_ref, w_ref, b_ref, o_ref):
    agg = p_ref[0] + p_ref[1]
    nd = lax.rsqrt(jnp.maximum(degi_ref[0, :, 0] + degi_ref[1, :, 0], 1.0))
    o_ref[...] = jnp.dot(agg * nd[:, None], w_ref[...],
                         preferred_element_type=jnp.float32) + b_ref[...]


def _row_spec(br):
    return pl.BlockSpec((br, D), lambda i: (i, 0))


def _part_spec(br):
    return pl.BlockSpec((NC, br, D), lambda i: (0, i, 0))


_w_spec = pl.BlockSpec((D, D), lambda i: (0, 0))
_b_spec = pl.BlockSpec((1, D), lambda i: (0, 0))

_scale_call = pl.pallas_call(
    _scale_body,
    grid=(GRID,),
    in_specs=[_row_spec(BR), _part_spec(BR)],
    out_specs=_row_spec(BR),
    out_shape=jax.ShapeDtypeStruct((N_PAD, D), jnp.float32),
)

_mid_call = pl.pallas_call(
    _mid_body,
    grid=(GRID,),
    in_specs=[_part_spec(BR), _part_spec(BR), _part_spec(BR), _w_spec, _b_spec],
    out_specs=_row_spec(BR),
    out_shape=jax.ShapeDtypeStruct((N_PAD, D), jnp.float32),
)

_fin_call = pl.pallas_call(
    _fin_body,
    grid=(GRIDF,),
    in_specs=[_part_spec(BRF), _part_spec(BRF), _w_spec, _b_spec],
    out_specs=_row_spec(BRF),
    out_shape=jax.ShapeDtypeStruct((N, D), jnp.float32),
)


def kernel(feat, edge_index, W1, b1, W2, b2):
    src = edge_index[0]
    dst = edge_index[1]
    pad = jnp.full((E_PAD - E,), N, jnp.int32)
    src2 = jnp.concatenate([src, pad]).reshape(NW * NCHUNK, C)
    dst2 = jnp.concatenate([dst, pad]).reshape(NW * NCHUNK, C)
    featp = jnp.zeros((N_PAD, D), feat.dtype).at[:N].set(feat)
    b1r = b1.reshape(1, D)
    b2r = b2.reshape(1, D)

    dego = _count_call(src2)
    degi = _count_call(dst2)
    h = _scale_call(featp, dego)
    parts = _spmm_call(h, src2, dst2)
    h1 = _mid_call(parts, degi, dego, W1, b1r)
    parts2 = _spmm_call(h1, src2, dst2)
    return _fin_call(parts2, degi, W2, b2r)


# speedup vs baseline: 3.3890x; 1.1119x over previous
import jax
import jax.numpy as jnp
from jax import lax
from jax.experimental import pallas as pl
from jax.experimental.pallas import tpu as pltpu
from jax.experimental.pallas import tpu_sc as plsc

N = 10000
E = 320000
D = 128

NC = 2
NS = 16
NW = NC * NS
C = 128
NCHUNK = 80
HC = NCHUNK // 2
EPT = NCHUNK * C
E_PAD = EPT * NW
N_PAD = 10240
RPT = N_PAD // NS
BR = 512
GRID = N_PAD // BR
BRF = 400
GRIDF = N // BRF

_MESH = plsc.VectorSubcoreMesh(
    core_axis_name="c", subcore_axis_name="s", num_cores=NC, num_subcores=NS
)


def _count_body(idx_hbm, tab_hbm, idx_vm, ones_vm, zero_vm, tab_sh):
    cid = lax.axis_index("c")
    sid = lax.axis_index("s")
    wid = cid * NS + sid

    pltpu.sync_copy(idx_hbm.at[pl.ds(wid * NCHUNK, NCHUNK)], idx_vm)

    one16 = jnp.full((16,), 1.0, jnp.float32)
    zero16 = jnp.zeros((16,), jnp.float32)
    for r in range(C):
        for q in range(D // 16):
            ones_vm[r, pl.ds(q * 16, 16)] = one16
    for r in range(16):
        for q in range(D // 16):
            zero_vm[r, pl.ds(q * 16, 16)] = zero16

    for k in range(RPT // 16):
        pltpu.sync_copy(zero_vm, tab_sh.at[pl.ds(sid * RPT + k * 16, 16)])
    plsc.subcore_barrier()

    def _chunk(j, _):
        pltpu.sync_copy(ones_vm, tab_sh.at[idx_vm.at[j]], add=True)
        return _

    lax.fori_loop(0, NCHUNK, _chunk, 0)
    plsc.subcore_barrier()

    sl = pl.ds(sid * RPT, RPT)
    pltpu.sync_copy(tab_sh.at[sl], tab_hbm.at[cid, sl])


_count_call = pl.kernel(
    _count_body,
    out_type=jax.ShapeDtypeStruct((NC, N_PAD, D), jnp.float32),
    mesh=_MESH,
    scratch_types=[
        pltpu.VMEM((NCHUNK, C), jnp.int32),
        pltpu.VMEM((C, D), jnp.float32),
        pltpu.VMEM((16, D), jnp.float32),
        pltpu.VMEM_SHARED((N_PAD, D), jnp.float32),
    ],
)


def _spmm_body(h_hbm, sidx_hbm, didx_hbm, out_hbm,
               sidx_vm, didx_vm, msg_vm, msg2_vm, zero_vm, acc_sh, sem, sem2):
    cid = lax.axis_index("c")
    sid = lax.axis_index("s")
    wid = cid * NS + sid

    zero16 = jnp.zeros((16,), jnp.float32)
    for r in range(16):
        for q in range(D // 16):
            zero_vm[r, pl.ds(q * 16, 16)] = zero16

    for k in range(RPT // 16):
        pltpu.sync_copy(zero_vm, acc_sh.at[pl.ds(sid * RPT + k * 16, 16)])
    plsc.subcore_barrier()

    def _half(p, _):
        row0 = wid * NCHUNK + p * HC
        pltpu.sync_copy(sidx_hbm.at[pl.ds(row0, HC)], sidx_vm)
        pltpu.sync_copy(didx_hbm.at[pl.ds(row0, HC)], didx_vm)
        pltpu.async_copy(h_hbm.at[sidx_vm.at[0]], msg_vm, sem)

        def _chunk(i, _2):
            j = 2 * i
            pltpu.async_copy(h_hbm.at[sidx_vm.at[j + 1]], msg2_vm, sem2)
            pltpu.make_async_copy(h_hbm.at[sidx_vm.at[j]], msg_vm, sem).wait()
            pltpu.sync_copy(msg_vm, acc_sh.at[didx_vm.at[j]], add=True)

            @pl.when(j + 2 < HC)
            def _g0():
                pltpu.async_copy(h_hbm.at[sidx_vm.at[j + 2]], msg_vm, sem)

            pltpu.make_async_copy(h_hbm.at[sidx_vm.at[j + 1]], msg2_vm, sem2).wait()
            pltpu.sync_copy(msg2_vm, acc_sh.at[didx_vm.at[j + 1]], add=True)
            return _2

        lax.fori_loop(0, HC // 2, _chunk, 0)
        return _

    lax.fori_loop(0, 2, _half, 0)
    plsc.subcore_barrier()

    sl = pl.ds(sid * RPT, RPT)
    pltpu.sync_copy(acc_sh.at[sl], out_hbm.at[cid, sl])


_spmm_call = pl.kernel(
    _spmm_body,
    out_type=jax.ShapeDtypeStruct((NC, N_PAD, D), jnp.float32),
    mesh=_MESH,
    scratch_types=[
        pltpu.VMEM((HC, C), jnp.int32),
        pltpu.VMEM((HC, C), jnp.int32),
        pltpu.VMEM((C, D), jnp.float32),
        pltpu.VMEM((C, D), jnp.float32),
        pltpu.VMEM((16, D), jnp.float32),
        pltpu.VMEM_SHARED((N_PAD, D), jnp.float32),
        pltpu.SemaphoreType.DMA,
        pltpu.SemaphoreType.DMA,
    ],
)


def _scale_body(feat_ref, dego_ref, o_ref):
    dg = dego_ref[0, :, 0] + dego_ref[1, :, 0]
    ns = lax.rsqrt(jnp.maximum(dg, 1.0))
    o_ref[...] = feat_ref[...] * ns[:, None]


def _mid_body(p_ref, degi_ref, dego_ref, w_ref, b_ref, o_ref):
    agg = p_ref[0] + p_ref[1]
    nd = lax.rsqrt(jnp.maximum(degi_ref[0, :, 0] + degi_ref[1, :, 0], 1.0))
    ns = lax.rsqrt(jnp.maximum(dego_ref[0, :, 0] + dego_ref[1, :, 0], 1.0))
    h = jnp.dot(agg * nd[:, None], w_ref[...],
                preferred_element_type=jnp.float32) + b_ref[...]
    o_ref[...] = h * ns[:, None]


def _fin_body(p_ref, degi_ref, w_ref, b_ref, o_ref):
    agg = p_ref[0] + p_ref[1]
    nd = lax.rsqrt(jnp.maximum(degi_ref[0, :, 0] + degi_ref[1, :, 0], 1.0))
    o_ref[...] = jnp.dot(agg * nd[:, None], w_ref[...],
                         preferred_element_type=jnp.float32) + b_ref[...]


def _row_spec(br):
    return pl.BlockSpec((br, D), lambda i: (i, 0))


def _part_spec(br):
    return pl.BlockSpec((NC, br, D), lambda i: (0, i, 0))


_w_spec = pl.BlockSpec((D, D), lambda i: (0, 0))
_b_spec = pl.BlockSpec((1, D), lambda i: (0, 0))

_scale_call = pl.pallas_call(
    _scale_body,
    grid=(GRID,),
    in_specs=[_row_spec(BR), _part_spec(BR)],
    out_specs=_row_spec(BR),
    out_shape=jax.ShapeDtypeStruct((N_PAD, D), jnp.float32),
)

_mid_call = pl.pallas_call(
    _mid_body,
    grid=(GRID,),
    in_specs=[_part_spec(BR), _part_spec(BR), _part_spec(BR), _w_spec, _b_spec],
    out_specs=_row_spec(BR),
    out_shape=jax.ShapeDtypeStruct((N_PAD, D), jnp.float32),
)

_fin_call = pl.pallas_call(
    _fin_body,
    grid=(GRIDF,),
    in_specs=[_part_spec(BRF), _part_spec(BRF), _w_spec, _b_spec],
    out_specs=_row_spec(BRF),
    out_shape=jax.ShapeDtypeStruct((N, D), jnp.float32),
)


def kernel(feat, edge_index, W1, b1, W2, b2):
    src = edge_index[0]
    dst = edge_index[1]
    pad = jnp.full((E_PAD - E,), N, jnp.int32)
    src2 = jnp.concatenate([src, pad]).reshape(NW * NCHUNK, C)
    dst2 = jnp.concatenate([dst, pad]).reshape(NW * NCHUNK, C)
    featp = jnp.zeros((N_PAD, D), feat.dtype).at[:N].set(feat)
    b1r = b1.reshape(1, D)
    b2r = b2.reshape(1, D)

    dego = _count_call(src2)
    degi = _count_call(dst2)
    h = _scale_call(featp, dego)
    parts = _spmm_call(h, src2, dst2)
    h1 = _mid_call(parts, degi, dego, W1, b1r)
    parts2 = _spmm_call(h1, src2, dst2)
    return _fin_call(parts2, degi, W2, b2r)
